# baseline (speedup 1.0000x reference)
import functools

import jax
import jax.numpy as jnp
from jax import lax
from jax.experimental import pallas as pl
from jax.experimental.pallas import tpu as pltpu
from jax.experimental.pallas import tpu_sc as plsc

BATCH = 16384
EMBED = 64
NROWS = 1000000
NC = 2
NS = 16
NW = NC * NS
BPW = BATCH // NW
CH = 128
NCH = BPW // CH
VL = 16

BLK = 2048


def _gather_body(uids_hbm, iids_hbm, utab_hbm, itab_hbm, u_out, i_out,
                 uidx_v, iidx_v, buf_a, buf_b, buf_c, sem_a, sem_b, sem_c):
    wid = lax.axis_index("s") * NC + lax.axis_index("c")
    base = wid * BPW
    for k in range(NCH):
        pltpu.sync_copy(uids_hbm.at[pl.ds(base + k * CH, CH)], uidx_v.at[k])
        pltpu.sync_copy(iids_hbm.at[pl.ds(base + k * CH, CH)], iidx_v.at[k])
    for idx_v in (uidx_v, iidx_v):
        for k in range(NCH):
            for g in range(CH // VL):
                sl = pl.ds(g * VL, VL)
                idx_v[k, sl] = idx_v[k, sl] >> 1

    bufs = (buf_a, buf_b, buf_c)
    sems = (sem_a, sem_b, sem_c)
    jobs = ([(utab_hbm, uidx_v, u_out, k) for k in range(NCH)]
            + [(itab_hbm, iidx_v, i_out, k) for k in range(NCH)])
    pending = {}
    for n, (tab, idxv, out, k) in enumerate(jobs):
        slot = n % 3
        if n >= 3:
            cp, pout, pk = pending[slot]
            cp.wait()
            pltpu.sync_copy(bufs[slot], pout.at[pl.ds(base + pk * CH, CH)])
        cp = pltpu.make_async_copy(tab.at[idxv.at[k]], bufs[slot], sems[slot])
        cp.start()
        pending[slot] = (cp, out, k)
    for n in range(3):
        slot = (len(jobs) - 3 + n) % 3
        cp, pout, pk = pending[slot]
        cp.wait()
        pltpu.sync_copy(bufs[slot], pout.at[pl.ds(base + pk * CH, CH)])


@functools.cache
def _sc_gather():
    return pl.kernel(
        _gather_body,
        mesh=plsc.VectorSubcoreMesh(core_axis_name="c", subcore_axis_name="s"),
        out_type=(
            jax.ShapeDtypeStruct((BATCH, 2 * EMBED), jnp.float32),
            jax.ShapeDtypeStruct((BATCH, 2 * EMBED), jnp.float32),
        ),
        scratch_types=[
            pltpu.VMEM((NCH, CH), jnp.int32),
            pltpu.VMEM((NCH, CH), jnp.int32),
            pltpu.VMEM((CH, 2 * EMBED), jnp.float32),
            pltpu.VMEM((CH, 2 * EMBED), jnp.float32),
            pltpu.VMEM((CH, 2 * EMBED), jnp.float32),
            pltpu.SemaphoreType.DMA,
            pltpu.SemaphoreType.DMA,
            pltpu.SemaphoreType.DMA,
        ],
    )


def _mlp_body(u_ref, i_ref, uid_ref, iid_ref, w1u_ref, w1i_ref, b1_ref,
              w2_ref, b2_ref, w3t_ref, b3_ref, out_ref):
    upar = (uid_ref[...] & 1)[:, None] == 1
    ipar = (iid_ref[...] & 1)[:, None] == 1
    u = jnp.where(upar, u_ref[:, EMBED:], u_ref[:, :EMBED])
    i = jnp.where(ipar, i_ref[:, EMBED:], i_ref[:, :EMBED])
    h = jnp.dot(u, w1u_ref[...], preferred_element_type=jnp.float32)
    h = h + jnp.dot(i, w1i_ref[...], preferred_element_type=jnp.float32)
    h = jnp.maximum(h + b1_ref[...], 0.0)
    h2 = jnp.dot(h, w2_ref[...], preferred_element_type=jnp.float32)
    h2 = jnp.maximum(h2 + b2_ref[...], 0.0)
    o = jnp.sum(h2 * w3t_ref[...], axis=1)
    out_ref[...] = o + b3_ref[0]


def _mlp(u, i, uids, iids, w1u, w1i, b1, w2, b2, w3t, b3):
    grid = (BATCH // BLK,)
    return pl.pallas_call(
        _mlp_body,
        grid=grid,
        in_specs=[
            pl.BlockSpec((BLK, 2 * EMBED), lambda g: (g, 0)),
            pl.BlockSpec((BLK, 2 * EMBED), lambda g: (g, 0)),
            pl.BlockSpec((BLK,), lambda g: (g,)),
            pl.BlockSpec((BLK,), lambda g: (g,)),
            pl.BlockSpec((EMBED, 512), lambda g: (0, 0)),
            pl.BlockSpec((EMBED, 512), lambda g: (0, 0)),
            pl.BlockSpec((1, 512), lambda g: (0, 0)),
            pl.BlockSpec((512, 256), lambda g: (0, 0)),
            pl.BlockSpec((1, 256), lambda g: (0, 0)),
            pl.BlockSpec((1, 256), lambda g: (0, 0)),
            pl.BlockSpec(memory_space=pltpu.SMEM),
        ],
        out_specs=pl.BlockSpec((BLK,), lambda g: (g,)),
        out_shape=jax.ShapeDtypeStruct((BATCH,), jnp.float32),
    )(u, i, uids, iids, w1u, w1i, b1, w2, b2, w3t, b3)


def kernel(user_ids, item_ids, user_table, item_table, W1, b1, W2, b2, W3, b3):
    uids = user_ids.astype(jnp.int32)
    iids = item_ids.astype(jnp.int32)
    u2, i2 = _sc_gather()(uids, iids,
                          user_table.reshape(NROWS // 2, 2 * EMBED),
                          item_table.reshape(NROWS // 2, 2 * EMBED))
    return _mlp(u2, i2, uids, iids, W1[:EMBED], W1[EMBED:],
                b1.reshape(1, 512), W2, b2.reshape(1, 256),
                W3.reshape(1, 256), b3)


# speedup vs baseline: 1.5529x; 1.5529x over previous
import functools

import jax
import jax.numpy as jnp
from jax import lax
from jax.experimental import pallas as pl
from jax.experimental.pallas import tpu as pltpu
from jax.experimental.pallas import tpu_sc as plsc

BATCH = 16384
EMBED = 64
NC = 2
NS = 16
NW = NC * NS
BPW = BATCH // NW

BLK = 2048


CHUNK = 256


def _gather_body(uids_hbm, iids_hbm, utab_hbm, itab_hbm, u_out, i_out,
                 uids_v, iids_v, buf_a, buf_b, buf_c,
                 sem_a, sem_b, sem_c):
    wid = lax.axis_index("s") * NC + lax.axis_index("c")
    base = wid * BPW
    pltpu.sync_copy(uids_hbm.at[pl.ds(base, BPW)], uids_v)
    pltpu.sync_copy(iids_hbm.at[pl.ds(base, BPW)], iids_v)

    def fire(ids_v, tab, buf, sem, off):
        def body(g, _):
            vec = ids_v[pl.ds(off + g * 16, 16)]
            for lane in range(16):
                r = vec[lane]
                pltpu.make_async_copy(tab.at[r], buf.at[g * 16 + lane],
                                      sem).start()
            return 0
        lax.fori_loop(0, CHUNK // 16, body, 0)

    def drain(buf, sem):
        pltpu.make_async_copy(utab_hbm.at[pl.ds(0, CHUNK)], buf, sem).wait()

    def copyout(buf, out, off):
        pltpu.sync_copy(buf, out.at[pl.ds(base + off, CHUNK)])

    fire(uids_v, utab_hbm, buf_a, sem_a, 0)
    fire(uids_v, utab_hbm, buf_b, sem_b, CHUNK)
    fire(iids_v, itab_hbm, buf_c, sem_c, 0)
    drain(buf_a, sem_a)
    copyout(buf_a, u_out, 0)
    fire(iids_v, itab_hbm, buf_a, sem_a, CHUNK)
    drain(buf_b, sem_b)
    copyout(buf_b, u_out, CHUNK)
    drain(buf_c, sem_c)
    copyout(buf_c, i_out, 0)
    drain(buf_a, sem_a)
    copyout(buf_a, i_out, CHUNK)


@functools.cache
def _sc_gather():
    return pl.kernel(
        _gather_body,
        mesh=plsc.VectorSubcoreMesh(core_axis_name="c", subcore_axis_name="s"),
        out_type=(
            jax.ShapeDtypeStruct((BATCH, EMBED), jnp.float32),
            jax.ShapeDtypeStruct((BATCH, EMBED), jnp.float32),
        ),
        scratch_types=[
            pltpu.VMEM((BPW,), jnp.int32),
            pltpu.VMEM((BPW,), jnp.int32),
            pltpu.VMEM((CHUNK, EMBED), jnp.float32),
            pltpu.VMEM((CHUNK, EMBED), jnp.float32),
            pltpu.VMEM((CHUNK, EMBED), jnp.float32),
            pltpu.SemaphoreType.DMA,
            pltpu.SemaphoreType.DMA,
            pltpu.SemaphoreType.DMA,
        ],
    )


def _mlp_body(u_ref, i_ref, w1u_ref, w1i_ref, b1_ref, w2_ref, b2_ref,
              w3t_ref, b3_ref, out_ref):
    h = jnp.dot(u_ref[...], w1u_ref[...], preferred_element_type=jnp.float32)
    h = h + jnp.dot(i_ref[...], w1i_ref[...], preferred_element_type=jnp.float32)
    h = jnp.maximum(h + b1_ref[...], 0.0)
    h2 = jnp.dot(h, w2_ref[...], preferred_element_type=jnp.float32)
    h2 = jnp.maximum(h2 + b2_ref[...], 0.0)
    o = jnp.sum(h2 * w3t_ref[...], axis=1)
    out_ref[...] = o + b3_ref[0]


def _mlp(u, i, w1u, w1i, b1, w2, b2, w3t, b3):
    grid = (BATCH // BLK,)
    return pl.pallas_call(
        _mlp_body,
        grid=grid,
        in_specs=[
            pl.BlockSpec((BLK, EMBED), lambda g: (g, 0)),
            pl.BlockSpec((BLK, EMBED), lambda g: (g, 0)),
            pl.BlockSpec((EMBED, 512), lambda g: (0, 0)),
            pl.BlockSpec((EMBED, 512), lambda g: (0, 0)),
            pl.BlockSpec((1, 512), lambda g: (0, 0)),
            pl.BlockSpec((512, 256), lambda g: (0, 0)),
            pl.BlockSpec((1, 256), lambda g: (0, 0)),
            pl.BlockSpec((1, 256), lambda g: (0, 0)),
            pl.BlockSpec(memory_space=pltpu.SMEM),
        ],
        out_specs=pl.BlockSpec((BLK,), lambda g: (g,)),
        out_shape=jax.ShapeDtypeStruct((BATCH,), jnp.float32),
    )(u, i, w1u, w1i, b1, w2, b2, w3t, b3)


def kernel(user_ids, item_ids, user_table, item_table, W1, b1, W2, b2, W3, b3):
    u, i = _sc_gather()(user_ids.astype(jnp.int32), item_ids.astype(jnp.int32),
                        user_table, item_table)
    return _mlp(u, i, W1[:EMBED], W1[EMBED:], b1.reshape(1, 512),
                W2, b2.reshape(1, 256), W3.reshape(1, 256), b3)
